# Initial kernel scaffold; baseline (speedup 1.0000x reference)
import jax
import jax.numpy as jnp
from jax.experimental import pallas as pl


def kernel(x, E0, E1, W1, b1, W2, b2):
    raise NotImplementedError("write your pallas kernel here")


# speedup vs baseline: 1.0621x; 1.0621x over previous
import jax
import jax.numpy as jnp
from jax.experimental import pallas as pl

P = 1024
O = 512
VOCAB = 200
RPAD = 256
R = 512


def _precompute_body(e0_ref, e1_ref, w1a_ref, w1b_ref, b1_ref, tt_ref):
    t0 = jnp.dot(e0_ref[...], w1a_ref[...], preferred_element_type=jnp.float32,
                 precision=jax.lax.Precision.HIGHEST)
    t1 = jnp.dot(e1_ref[...], w1b_ref[...], preferred_element_type=jnp.float32,
                 precision=jax.lax.Precision.HIGHEST)
    tt_ref[0:RPAD, :] = t0
    tt_ref[RPAD:2 * RPAD, :] = t1 + b1_ref[...]


def _fused_body(i0_ref, i1_ref, tt_ref, w2_ref, b2_ref, out_ref):
    iv = i0_ref[0]
    jv = i1_ref[0]
    col = jax.lax.broadcasted_iota(jnp.int32, (R, 2 * RPAD), 1)
    oh = ((col == iv) | (col == jv)).astype(jnp.float32)
    g = jnp.dot(oh, tt_ref[...], preferred_element_type=jnp.float32,
                precision=jax.lax.Precision.HIGHEST)
    h = jnp.maximum(g, 0.0)
    out_ref[...] = jnp.dot(h, w2_ref[...], preferred_element_type=jnp.float32,
                           precision=jax.lax.Precision.HIGHEST) + b2_ref[...]


def kernel(x, E0, E1, W1, b1, W2, b2):
    B = x.shape[0]
    nsteps = B // R

    e0p = jnp.pad(E0, ((0, RPAD - VOCAB), (0, 0)))
    e1p = jnp.pad(E1, ((0, RPAD - VOCAB), (0, 0)))
    w1a = W1[:P]
    w1b = W1[P:]

    tt = pl.pallas_call(
        _precompute_body,
        out_shape=jax.ShapeDtypeStruct((2 * RPAD, P), jnp.float32),
    )(e0p, e1p, w1a, w1b, b1.reshape(1, P))

    i0 = x[:, 0].astype(jnp.int32).reshape(nsteps, R, 1)
    i1 = (x[:, 1].astype(jnp.int32) + RPAD).reshape(nsteps, R, 1)

    out = pl.pallas_call(
        _fused_body,
        grid=(nsteps,),
        in_specs=[
            pl.BlockSpec((1, R, 1), lambda i: (i, 0, 0)),
            pl.BlockSpec((1, R, 1), lambda i: (i, 0, 0)),
            pl.BlockSpec((2 * RPAD, P), lambda i: (0, 0)),
            pl.BlockSpec((P, O), lambda i: (0, 0)),
            pl.BlockSpec((1, O), lambda i: (0, 0)),
        ],
        out_specs=pl.BlockSpec((R, O), lambda i: (i, 0)),
        out_shape=jax.ShapeDtypeStruct((B, O), jnp.float32),
    )(i0, i1, tt, W2, b2.reshape(1, O))
    return out


# speedup vs baseline: 3.0011x; 2.8255x over previous
import jax
import jax.numpy as jnp
from jax.experimental import pallas as pl

P = 1024
O = 512
VOCAB = 200
RPAD = 256
R = 512


def _precompute_body(e0_ref, e1_ref, w1a_ref, w1b_ref, b1_ref, tt_ref):
    t0 = jnp.dot(e0_ref[...], w1a_ref[...], preferred_element_type=jnp.float32,
                 precision=jax.lax.Precision.HIGHEST)
    t1 = jnp.dot(e1_ref[...], w1b_ref[...], preferred_element_type=jnp.float32,
                 precision=jax.lax.Precision.HIGHEST)
    tt_ref[0:RPAD, :] = t0.astype(jnp.bfloat16)
    tt_ref[RPAD:2 * RPAD, :] = (t1 + b1_ref[...]).astype(jnp.bfloat16)


def _fused_body(i0_ref, i1_ref, tt_ref, w2_ref, b2_ref, out_ref):
    iv = i0_ref[0]
    jv = i1_ref[0]
    col = jax.lax.broadcasted_iota(jnp.int32, (R, 2 * RPAD), 1)
    oh = ((col == iv) | (col == jv)).astype(jnp.bfloat16)
    g = jnp.dot(oh, tt_ref[...], preferred_element_type=jnp.float32)
    h = jnp.maximum(g, 0.0).astype(jnp.bfloat16)
    out_ref[...] = jnp.dot(h, w2_ref[...], preferred_element_type=jnp.float32) + b2_ref[...]


def kernel(x, E0, E1, W1, b1, W2, b2):
    B = x.shape[0]
    nsteps = B // R

    e0p = jnp.pad(E0, ((0, RPAD - VOCAB), (0, 0)))
    e1p = jnp.pad(E1, ((0, RPAD - VOCAB), (0, 0)))
    w1a = W1[:P]
    w1b = W1[P:]

    tt = pl.pallas_call(
        _precompute_body,
        out_shape=jax.ShapeDtypeStruct((2 * RPAD, P), jnp.bfloat16),
    )(e0p, e1p, w1a, w1b, b1.reshape(1, P))

    i0 = x[:, 0].astype(jnp.int32).reshape(nsteps, R, 1)
    i1 = (x[:, 1].astype(jnp.int32) + RPAD).reshape(nsteps, R, 1)

    out = pl.pallas_call(
        _fused_body,
        grid=(nsteps,),
        in_specs=[
            pl.BlockSpec((1, R, 1), lambda i: (i, 0, 0)),
            pl.BlockSpec((1, R, 1), lambda i: (i, 0, 0)),
            pl.BlockSpec((2 * RPAD, P), lambda i: (0, 0)),
            pl.BlockSpec((P, O), lambda i: (0, 0)),
            pl.BlockSpec((1, O), lambda i: (0, 0)),
        ],
        out_specs=pl.BlockSpec((R, O), lambda i: (i, 0)),
        out_shape=jax.ShapeDtypeStruct((B, O), jnp.float32),
    )(i0, i1, tt, W2.astype(jnp.bfloat16), b2.reshape(1, O))
    return out
